# Initial kernel scaffold; baseline (speedup 1.0000x reference)
import jax
import jax.numpy as jnp
from jax.experimental import pallas as pl


def kernel(x, table):
    raise NotImplementedError("write your pallas kernel here")


# speedup vs baseline: 3.5470x; 3.5470x over previous
import functools

import jax
import jax.numpy as jnp
from jax import lax
from jax.experimental import pallas as pl
from jax.experimental.pallas import tpu as pltpu
from jax.experimental.pallas import tpu_sc as plsc

EMBED_DIM = 64
BATCH = 4096
SEQ = 200
NC = 2
NS = 16
NW = NC * NS
B = BATCH * SEQ
CHUNK = 128
B_PER_W = B // NW
CHUNKS = B_PER_W // CHUNK

_mesh = plsc.VectorSubcoreMesh(core_axis_name="c", subcore_axis_name="s")


@functools.partial(
    pl.kernel,
    out_type=jax.ShapeDtypeStruct((NW * CHUNKS, CHUNK, EMBED_DIM), jnp.float32),
    mesh=_mesh,
    scratch_types=[
        pltpu.VMEM((CHUNKS, CHUNK), jnp.int32),
        pltpu.VMEM((CHUNK, EMBED_DIM), jnp.float32),
        pltpu.SemaphoreType.DMA,
    ],
    compiler_params=pltpu.CompilerParams(use_tc_tiling_on_sc=False),
)
def _embed_gather(table_hbm, x_hbm, out_hbm, idx_v, rows_v, sem):
    wid = lax.axis_index("s") * NC + lax.axis_index("c")
    pltpu.sync_copy(x_hbm.at[wid], idx_v)

    def step(j, carry):
        pltpu.async_copy(table_hbm.at[idx_v.at[j]], rows_v, sem).wait()
        pltpu.sync_copy(rows_v, out_hbm.at[wid * CHUNKS + j])
        return carry

    lax.fori_loop(0, CHUNKS, step, 0)


def kernel(x, table):
    xr = x.astype(jnp.int32).reshape(NW, CHUNKS, CHUNK)
    out = _embed_gather(table, xr)
    return out.reshape(BATCH, SEQ, EMBED_DIM)


# speedup vs baseline: 4.2599x; 1.2010x over previous
import functools

import jax
import jax.numpy as jnp
from jax import lax
from jax.experimental import pallas as pl
from jax.experimental.pallas import tpu as pltpu
from jax.experimental.pallas import tpu_sc as plsc

EMBED_DIM = 64
BATCH = 4096
SEQ = 200
NC = 2
NS = 16
NW = NC * NS
B = BATCH * SEQ
CHUNK = 128
B_PER_W = B // NW
CHUNKS = B_PER_W // CHUNK
NBUF = 8
NGROUPS = CHUNKS // NBUF

_mesh = plsc.VectorSubcoreMesh(core_axis_name="c", subcore_axis_name="s")


@functools.partial(
    pl.kernel,
    out_type=jax.ShapeDtypeStruct((NW * CHUNKS, CHUNK, EMBED_DIM), jnp.float32),
    mesh=_mesh,
    scratch_types=[
        pltpu.VMEM((CHUNKS, CHUNK), jnp.int32),
        pltpu.VMEM((NBUF, CHUNK, EMBED_DIM), jnp.float32),
    ]
    + [pltpu.SemaphoreType.DMA] * (2 * NBUF),
    compiler_params=pltpu.CompilerParams(use_tc_tiling_on_sc=False),
)
def _embed_gather(table_hbm, x_hbm, out_hbm, idx_v, rows_v, *sems):
    gsem = sems[:NBUF]
    wsem = sems[NBUF:]
    wid = lax.axis_index("s") * NC + lax.axis_index("c")
    pltpu.sync_copy(x_hbm.at[wid], idx_v)
    out_base = wid * CHUNKS

    for b in range(NBUF):
        pltpu.async_copy(table_hbm.at[idx_v.at[b]], rows_v.at[b], gsem[b])

    @pl.loop(0, NGROUPS)
    def _ring(grp):
        j0 = grp * NBUF
        for b in range(NBUF):
            pltpu.make_async_copy(
                table_hbm.at[idx_v.at[j0 + b]], rows_v.at[b], gsem[b]
            ).wait()
            pltpu.async_copy(rows_v.at[b], out_hbm.at[out_base + j0 + b], wsem[b])
        for b in range(NBUF):
            pltpu.make_async_copy(
                rows_v.at[b], out_hbm.at[out_base + j0 + b], wsem[b]
            ).wait()

            @pl.when(grp < NGROUPS - 1)
            def _():
                pltpu.async_copy(
                    table_hbm.at[idx_v.at[j0 + NBUF + b]], rows_v.at[b], gsem[b]
                )


def kernel(x, table):
    xr = x.astype(jnp.int32).reshape(NW, CHUNKS, CHUNK)
    out = _embed_gather(table, xr)
    return out.reshape(BATCH, SEQ, EMBED_DIM)
